# baseline (speedup 1.0000x reference)
import dataclasses
import functools

import numpy as np
import jax
import jax.numpy as jnp
from jax import lax
from jax.experimental import pallas as pl
from jax.experimental.pallas import tpu as pltpu
from jax.experimental.pallas import tpu_sc as plsc

NU = 5000
NI = 5000
E = 320000
D = 128

NTILES = 16
CHUNK = E // NTILES
HSTRIDE = 5120
HWORDS = 16 * HSTRIDE
PAD = 5120
ROWS_PER_TILE = PAD // NTILES
BLK = 125
SUB = 16
STAGE = SUB * BLK
NCH = CHUNK // STAGE

_mesh = plsc.VectorSubcoreMesh(core_axis_name="c", subcore_axis_name="s")

_sc_params = pltpu.CompilerParams()
if "needs_layout_passes" in pltpu.CompilerParams.__dataclass_fields__:
    _sc_params = dataclasses.replace(_sc_params, needs_layout_passes=False)


def _hist_body(uidx_hbm, iidx_hbm, zeros_hbm, out_u, out_i, idx_v, hist_v):
    cid = lax.axis_index("c")
    sid = lax.axis_index("s")
    base = sid * CHUNK
    lane_off = lax.iota(jnp.int32, 16) * HSTRIDE
    ones16 = jnp.ones((16,), jnp.float32)

    def one_side(src_hbm, out_hbm):
        pltpu.sync_copy(src_hbm.at[pl.ds(base, CHUNK)], idx_v)
        pltpu.sync_copy(zeros_hbm, hist_v)

        @pl.loop(0, CHUNK, step=80)
        def _(i):
            for j in range(5):
                idx = idx_v[pl.ds(i + 16 * j, 16)]
                plsc.addupdate_scatter(hist_v, [idx + lane_off], ones16)

        pltpu.sync_copy(hist_v, out_hbm.at[sid])

    @pl.when(cid == 0)
    def _():
        one_side(uidx_hbm, out_u)

    @pl.when(cid == 1)
    def _():
        one_side(iidx_hbm, out_i)


_hist_kernel = pl.kernel(
    _hist_body,
    out_type=[
        jax.ShapeDtypeStruct((NTILES, HWORDS), jnp.float32),
        jax.ShapeDtypeStruct((NTILES, HWORDS), jnp.float32),
    ],
    mesh=_mesh,
    scratch_types=[
        pltpu.VMEM((CHUNK,), jnp.int32),
        pltpu.VMEM((HWORDS,), jnp.float32),
    ],
    compiler_params=_sc_params,
)


def _prescale_body(hu_ref, hi_ref, uep_ref, iep_ref,
                   cat_ref, aub_ref, aib_ref):
    deg_u = jnp.sum(hu_ref[...].reshape(16 * NTILES, HSTRIDE), axis=0)[:NU]
    deg_i = jnp.sum(hi_ref[...].reshape(16 * NTILES, HSTRIDE), axis=0)[:NI]
    a_u = lax.rsqrt(jnp.maximum(deg_u, 1.0))
    a_i = lax.rsqrt(jnp.maximum(deg_i, 1.0))
    aub = jnp.broadcast_to(a_u[:, None], (NU, D))
    aib = jnp.broadcast_to(a_i[:, None], (NI, D))
    aub_ref[...] = aub
    aib_ref[...] = aib
    cat_ref[0:NI, 0:D] = (iep_ref[...] * aib).astype(jnp.bfloat16)
    cat_ref[NI:NI + NU, 0:D] = (uep_ref[...] * aub).astype(jnp.bfloat16)
    cat_ref[:, D:2 * D] = jnp.zeros((NI + NU, D), jnp.bfloat16)


def _prescale(hu, hi, u_emb, i_emb):
    return pl.pallas_call(
        _prescale_body,
        out_shape=[
            jax.ShapeDtypeStruct((NI + NU, 2 * D), jnp.bfloat16),
            jax.ShapeDtypeStruct((NU, D), jnp.float32),
            jax.ShapeDtypeStruct((NI, D), jnp.float32),
        ],
    )(hu, hi, u_emb, i_emb)


UNROLL = 25

_PERM = np.concatenate([
    np.stack([np.arange(g * 32, g * 32 + 16),
              np.arange(g * 32 + 16, g * 32 + 32)], axis=1).reshape(-1)
    for g in range(4)])


def _scatter_body(emb_cat_hbm, sidx2_hbm, didx2_hbm, w_hbm, zrow_hbm,
                  out_hbm, sidx_s, didx_s, w_s, vin0, vin1, vf0, vf1,
                  acc_sh, semg0, semg1, sems0, sems1):
    cid = lax.axis_index("c")
    sid = lax.axis_index("s")

    def mulscatter(vin, vf, b):
        @pl.loop(0, BLK, step=UNROLL)
        def _(e):
            for j in range(UNROLL):
                wv = plsc.load_gather(
                    w_s,
                    [jnp.broadcast_to(b * BLK + e + (j + 16), (16,)).astype(jnp.int32)])
                for g in range(4):
                    vi = vin[e + j, pl.ds(g * 16, 16)]
                    vb = plsc.bitcast(vi, jnp.bfloat16)
                    a_, b_ = plsc.unpack(vb, format=plsc.PackFormat.INTERLEAVED)
                    vf[e + j, pl.ds(g * 32, 16)] = a_.astype(jnp.float32) * wv
                    vf[e + j, pl.ds(g * 32 + 16, 16)] = b_.astype(jnp.float32) * wv
        pltpu.sync_copy(vf, acc_sh.at[didx_s.at[b]], add=True)

    pltpu.sync_copy(zrow_hbm,
                    acc_sh.at[pl.ds(sid * ROWS_PER_TILE, ROWS_PER_TILE), :])
    plsc.subcore_barrier()

    @pl.loop(0, NCH)
    def _(ch):
        row0 = sid * (CHUNK // BLK) + ch * SUB
        pltpu.sync_copy(sidx2_hbm.at[cid].at[pl.ds(row0, SUB), :], sidx_s)
        pltpu.sync_copy(didx2_hbm.at[cid].at[pl.ds(row0, SUB), :], didx_s)
        pltpu.sync_copy(w_hbm.at[pl.ds(sid * CHUNK + ch * STAGE, STAGE)],
                        w_s.at[pl.ds(16, STAGE)])
        pltpu.async_copy(emb_cat_hbm.at[sidx_s.at[0]], vin0, semg0)

        @pl.loop(0, SUB // 2 - 1)
        def _(k):
            b = 2 * k
            pltpu.make_async_copy(emb_cat_hbm.at[sidx_s.at[b]], vin0, semg0).wait()
            pltpu.async_copy(emb_cat_hbm.at[sidx_s.at[b + 1]], vin1, semg1)
            mulscatter(vin0, vf0, b)
            pltpu.make_async_copy(emb_cat_hbm.at[sidx_s.at[b + 1]], vin1, semg1).wait()
            pltpu.async_copy(emb_cat_hbm.at[sidx_s.at[b + 2]], vin0, semg0)
            mulscatter(vin1, vf1, b + 1)

        b_last = SUB - 2
        pltpu.make_async_copy(emb_cat_hbm.at[sidx_s.at[b_last]], vin0, semg0).wait()
        pltpu.async_copy(emb_cat_hbm.at[sidx_s.at[b_last + 1]], vin1, semg1)
        mulscatter(vin0, vf0, b_last)
        pltpu.make_async_copy(emb_cat_hbm.at[sidx_s.at[b_last + 1]], vin1, semg1).wait()
        mulscatter(vin1, vf1, b_last + 1)

    plsc.subcore_barrier()
    pltpu.sync_copy(
        acc_sh.at[pl.ds(sid * ROWS_PER_TILE, ROWS_PER_TILE), :],
        out_hbm.at[cid].at[pl.ds(sid * ROWS_PER_TILE, ROWS_PER_TILE), :],
    )


_scatter_kernel = pl.kernel(
    _scatter_body,
    out_type=jax.ShapeDtypeStruct((2, PAD, D), jnp.float32),
    mesh=_mesh,
    scratch_types=[
        pltpu.VMEM((SUB, BLK), jnp.int32),
        pltpu.VMEM((SUB, BLK), jnp.int32),
        pltpu.VMEM((16 + STAGE,), jnp.float32),
        pltpu.VMEM((BLK, D), jnp.int32),
        pltpu.VMEM((BLK, D), jnp.int32),
        pltpu.VMEM((BLK, D), jnp.float32),
        pltpu.VMEM((BLK, D), jnp.float32),
        pltpu.VMEM_SHARED((PAD, D), jnp.float32),
        pltpu.SemaphoreType.DMA,
        pltpu.SemaphoreType.DMA,
        pltpu.SemaphoreType.DMA,
        pltpu.SemaphoreType.DMA,
    ],
    compiler_params=_sc_params,
)


def _finish_body(msg_ref, ab_ref, emb_ref, w_ref, out_ref):
    x = msg_ref[...] * ab_ref[...] + emb_ref[...]
    y = lax.dot_general(x, w_ref[...], (((1,), (1,)), ((), ())),
                        preferred_element_type=jnp.float32)
    out_ref[...] = jnp.maximum(y, 0.0)


def _finish(msg, ab, emb, W):
    return pl.pallas_call(
        _finish_body,
        out_shape=jax.ShapeDtypeStruct(emb.shape, jnp.float32),
    )(msg, ab, emb, W)


@jax.jit
def kernel(u_emb, i_emb, edge_index, weights, W_u, W_i):
    user_idx = edge_index[0].astype(jnp.int32)
    item_idx = edge_index[1].astype(jnp.int32)
    zeros_hist = jnp.zeros((HWORDS,), jnp.float32)
    zeros_rows = jnp.zeros((ROWS_PER_TILE, D), jnp.float32)
    uidx2 = user_idx.reshape(E // BLK, BLK)
    iidx2 = item_idx.reshape(E // BLK, BLK)
    sidx2 = jnp.stack([iidx2, uidx2 + NI])
    didx2 = jnp.stack([uidx2, iidx2])

    hu, hi = _hist_kernel(user_idx, item_idx, zeros_hist)
    perm = jnp.asarray(_PERM)
    cat_bf, aub, aib = _prescale(hu, hi, u_emb[:, perm], i_emb[:, perm])
    cat = lax.bitcast_convert_type(cat_bf.reshape(NI + NU, D, 2), jnp.int32)
    msg = _scatter_kernel(cat, sidx2, didx2, weights, zeros_rows)
    new_u = _finish(msg[0, :NU], aub, u_emb, W_u)
    new_i = _finish(msg[1, :NI], aib, i_emb, W_i)
    return (new_u, new_i)


# speedup vs baseline: 1.2567x; 1.2567x over previous
import dataclasses
import functools

import jax
import jax.numpy as jnp
from jax import lax
from jax.experimental import pallas as pl
from jax.experimental.pallas import tpu as pltpu
from jax.experimental.pallas import tpu_sc as plsc

NU = 5000
NI = 5000
E = 320000
D = 128

NTILES = 16
CHUNK = E // NTILES
HSTRIDE = 5120
HWORDS = 16 * HSTRIDE
PAD = 5120
ROWS_PER_TILE = PAD // NTILES
BLK = 125
SUB = 16
STAGE = SUB * BLK
NCH = CHUNK // STAGE

_mesh = plsc.VectorSubcoreMesh(core_axis_name="c", subcore_axis_name="s")

_sc_params = pltpu.CompilerParams()
if "needs_layout_passes" in pltpu.CompilerParams.__dataclass_fields__:
    _sc_params = dataclasses.replace(_sc_params, needs_layout_passes=False)


def _hist_body(uidx_hbm, iidx_hbm, zeros_hbm, out_u, out_i, idx_v, hist_v):
    cid = lax.axis_index("c")
    sid = lax.axis_index("s")
    base = sid * CHUNK
    lane_off = lax.iota(jnp.int32, 16) * HSTRIDE
    ones16 = jnp.ones((16,), jnp.float32)

    def one_side(src_hbm, out_hbm):
        pltpu.sync_copy(src_hbm.at[pl.ds(base, CHUNK)], idx_v)
        pltpu.sync_copy(zeros_hbm, hist_v)

        @pl.loop(0, CHUNK, step=80)
        def _(i):
            for j in range(5):
                idx = idx_v[pl.ds(i + 16 * j, 16)]
                plsc.addupdate_scatter(hist_v, [idx + lane_off], ones16)

        pltpu.sync_copy(hist_v, out_hbm.at[sid])

    @pl.when(cid == 0)
    def _():
        one_side(uidx_hbm, out_u)

    @pl.when(cid == 1)
    def _():
        one_side(iidx_hbm, out_i)


_hist_kernel = pl.kernel(
    _hist_body,
    out_type=[
        jax.ShapeDtypeStruct((NTILES, HWORDS), jnp.float32),
        jax.ShapeDtypeStruct((NTILES, HWORDS), jnp.float32),
    ],
    mesh=_mesh,
    scratch_types=[
        pltpu.VMEM((CHUNK,), jnp.int32),
        pltpu.VMEM((HWORDS,), jnp.float32),
    ],
    compiler_params=_sc_params,
)


def _prescale_body(hu_ref, hi_ref, uep_ref, iep_ref,
                   cat_ref, aub_ref, aib_ref):
    deg_u = jnp.sum(hu_ref[...].reshape(16 * NTILES, HSTRIDE), axis=0)[:NU]
    deg_i = jnp.sum(hi_ref[...].reshape(16 * NTILES, HSTRIDE), axis=0)[:NI]
    a_u = lax.rsqrt(jnp.maximum(deg_u, 1.0))
    a_i = lax.rsqrt(jnp.maximum(deg_i, 1.0))
    aub = jnp.broadcast_to(a_u[:, None], (NU, D))
    aib = jnp.broadcast_to(a_i[:, None], (NI, D))
    aub_ref[...] = aub
    aib_ref[...] = aib
    cat_ref[0:NI, :] = iep_ref[...] * aib
    cat_ref[NI:NI + NU, :] = uep_ref[...] * aub


def _prescale(hu, hi, u_emb, i_emb):
    return pl.pallas_call(
        _prescale_body,
        out_shape=[
            jax.ShapeDtypeStruct((NI + NU, D), jnp.float32),
            jax.ShapeDtypeStruct((NU, D), jnp.float32),
            jax.ShapeDtypeStruct((NI, D), jnp.float32),
        ],
    )(hu, hi, u_emb, i_emb)


UNROLL = 25


def _scatter_body(emb_cat_hbm, sidx2_hbm, didx2_hbm, w_hbm, zrow_hbm,
                  out_hbm, sidx_s, didx_s, w_s, vin0, vin1, vf0, vf1,
                  acc_sh, semg0, semg1, sems0, sems1):
    cid = lax.axis_index("c")
    sid = lax.axis_index("s")

    def mulscatter(vin, vf, b):
        del vf
        @pl.loop(0, BLK, step=UNROLL)
        def _(e):
            for j in range(UNROLL):
                wv = plsc.load_gather(
                    w_s,
                    [jnp.broadcast_to(b * BLK + e + (j + 16), (16,)).astype(jnp.int32)])
                for c in range(D // 16):
                    sl = pl.ds(c * 16, 16)
                    vin[e + j, sl] = vin[e + j, sl] * wv
        pltpu.sync_copy(vin, acc_sh.at[didx_s.at[b]], add=True)

    pltpu.sync_copy(zrow_hbm,
                    acc_sh.at[pl.ds(sid * ROWS_PER_TILE, ROWS_PER_TILE), :])
    plsc.subcore_barrier()

    @pl.loop(0, NCH)
    def _(ch):
        row0 = sid * (CHUNK // BLK) + ch * SUB
        pltpu.sync_copy(sidx2_hbm.at[cid].at[pl.ds(row0, SUB), :], sidx_s)
        pltpu.sync_copy(didx2_hbm.at[cid].at[pl.ds(row0, SUB), :], didx_s)
        pltpu.sync_copy(w_hbm.at[pl.ds(sid * CHUNK + ch * STAGE, STAGE)],
                        w_s.at[pl.ds(16, STAGE)])
        pltpu.async_copy(emb_cat_hbm.at[sidx_s.at[0]], vin0, semg0)

        @pl.loop(0, SUB // 2 - 1)
        def _(k):
            b = 2 * k
            pltpu.make_async_copy(emb_cat_hbm.at[sidx_s.at[b]], vin0, semg0).wait()
            pltpu.async_copy(emb_cat_hbm.at[sidx_s.at[b + 1]], vin1, semg1)
            mulscatter(vin0, vf0, b)
            pltpu.make_async_copy(emb_cat_hbm.at[sidx_s.at[b + 1]], vin1, semg1).wait()
            pltpu.async_copy(emb_cat_hbm.at[sidx_s.at[b + 2]], vin0, semg0)
            mulscatter(vin1, vf1, b + 1)

        b_last = SUB - 2
        pltpu.make_async_copy(emb_cat_hbm.at[sidx_s.at[b_last]], vin0, semg0).wait()
        pltpu.async_copy(emb_cat_hbm.at[sidx_s.at[b_last + 1]], vin1, semg1)
        mulscatter(vin0, vf0, b_last)
        pltpu.make_async_copy(emb_cat_hbm.at[sidx_s.at[b_last + 1]], vin1, semg1).wait()
        mulscatter(vin1, vf1, b_last + 1)

    plsc.subcore_barrier()
    pltpu.sync_copy(
        acc_sh.at[pl.ds(sid * ROWS_PER_TILE, ROWS_PER_TILE), :],
        out_hbm.at[cid].at[pl.ds(sid * ROWS_PER_TILE, ROWS_PER_TILE), :],
    )


_scatter_kernel = pl.kernel(
    _scatter_body,
    out_type=jax.ShapeDtypeStruct((2, PAD, D), jnp.float32),
    mesh=_mesh,
    scratch_types=[
        pltpu.VMEM((SUB, BLK), jnp.int32),
        pltpu.VMEM((SUB, BLK), jnp.int32),
        pltpu.VMEM((16 + STAGE,), jnp.float32),
        pltpu.VMEM((BLK, D), jnp.float32),
        pltpu.VMEM((BLK, D), jnp.float32),
        pltpu.VMEM((BLK, D), jnp.float32),
        pltpu.VMEM((BLK, D), jnp.float32),
        pltpu.VMEM_SHARED((PAD, D), jnp.float32),
        pltpu.SemaphoreType.DMA,
        pltpu.SemaphoreType.DMA,
        pltpu.SemaphoreType.DMA,
        pltpu.SemaphoreType.DMA,
    ],
    compiler_params=_sc_params,
)


def _finish_body(msg_ref, ab_ref, emb_ref, w_ref, out_ref):
    x = msg_ref[...] * ab_ref[...] + emb_ref[...]
    y = lax.dot_general(x, w_ref[...], (((1,), (1,)), ((), ())),
                        preferred_element_type=jnp.float32)
    out_ref[...] = jnp.maximum(y, 0.0)


def _finish(msg, ab, emb, W):
    return pl.pallas_call(
        _finish_body,
        out_shape=jax.ShapeDtypeStruct(emb.shape, jnp.float32),
    )(msg, ab, emb, W)


@jax.jit
def kernel(u_emb, i_emb, edge_index, weights, W_u, W_i):
    user_idx = edge_index[0].astype(jnp.int32)
    item_idx = edge_index[1].astype(jnp.int32)
    zeros_hist = jnp.zeros((HWORDS,), jnp.float32)
    zeros_rows = jnp.zeros((ROWS_PER_TILE, D), jnp.float32)
    uidx2 = user_idx.reshape(E // BLK, BLK)
    iidx2 = item_idx.reshape(E // BLK, BLK)
    sidx2 = jnp.stack([iidx2, uidx2 + NI])
    didx2 = jnp.stack([uidx2, iidx2])

    hu, hi = _hist_kernel(user_idx, item_idx, zeros_hist)
    cat, aub, aib = _prescale(hu, hi, u_emb, i_emb)
    msg = _scatter_kernel(cat, sidx2, didx2, weights, zeros_rows)
    new_u = _finish(msg[0, :NU], aub, u_emb, W_u)
    new_i = _finish(msg[1, :NI], aib, i_emb, W_i)
    return (new_u, new_i)
